# baseline (speedup 1.0000x reference)
import functools

import jax
import jax.numpy as jnp
from jax import lax
from jax.experimental import pallas as pl
from jax.experimental.pallas import tpu as pltpu
from jax.experimental.pallas import tpu_sc as plsc

ACQ = 1
B, A, S, C = 64, 4, 4096, 2
ROW = S * C

_NC = 2
_NS = 16
_ROWS_PER_W = B // (_NC * _NS)


def _copy_body(in_hbm, out_hbm, sem0, sem1):
    wid = lax.axis_index("s") * _NC + lax.axis_index("c")
    b = wid * _ROWS_PER_W
    c0 = pltpu.make_async_copy(in_hbm.at[b, ACQ], out_hbm.at[b], sem0)
    c1 = pltpu.make_async_copy(in_hbm.at[b + 1, ACQ], out_hbm.at[b + 1], sem1)
    c0.start()
    c1.start()
    c0.wait()
    c1.wait()


_copy = functools.partial(
    pl.kernel,
    out_type=jax.ShapeDtypeStruct((B, S, C), jnp.float32),
    mesh=plsc.VectorSubcoreMesh(core_axis_name="c", subcore_axis_name="s"),
    scratch_types=[pltpu.SemaphoreType.DMA, pltpu.SemaphoreType.DMA],
)(_copy_body)


@jax.jit
def kernel(inputs):
    return _copy(inputs)


# speedup vs baseline: 53.8662x; 53.8662x over previous
import functools

import jax
import jax.numpy as jnp
from jax import lax
from jax.experimental import pallas as pl
from jax.experimental.pallas import tpu as pltpu
from jax.experimental.pallas import tpu_sc as plsc

ACQ = 1
B, A, S, C = 64, 4, 4096, 2
LANES = 128
SB = S // LANES
ROWS = S * C // LANES

_NC = 2
_HALF = B // _NC


def _copy_body(in_hbm, out_hbm, sem):
    cid = lax.axis_index("c")
    b = cid * _HALF
    c0 = pltpu.make_async_copy(
        in_hbm.at[pl.ds(b, _HALF), ACQ], out_hbm.at[pl.ds(b, _HALF)], sem
    )
    c0.start()
    c0.wait()


_copy = functools.partial(
    pl.kernel,
    out_type=jax.ShapeDtypeStruct((B, ROWS, LANES), jnp.float32),
    mesh=plsc.ScalarSubcoreMesh(axis_name="c"),
    scratch_types=[pltpu.SemaphoreType.DMA],
)(_copy_body)


@jax.jit
def kernel(inputs):
    x = inputs.reshape(B, A, SB, LANES, C)
    x = x.transpose(0, 1, 2, 4, 3).reshape(B, A, ROWS, LANES)
    out = _copy(x)
    out = out.reshape(B, SB, C, LANES).transpose(0, 1, 3, 2)
    return out.reshape(B, S, C)


# speedup vs baseline: 67.6321x; 1.2556x over previous
import jax
import jax.numpy as jnp
from jax.experimental import pallas as pl
from jax.experimental.pallas import tpu as pltpu

ACQ = 1
B, A, S, C = 64, 4, 4096, 2
LANES = 128
SB = S // LANES
ROWS = S * C // LANES


def _copy_body(in_hbm, out_hbm, sem):
    pltpu.make_async_copy(in_hbm.at[:, ACQ], out_hbm, sem).start()
    pltpu.make_async_copy(in_hbm.at[:, ACQ], out_hbm, sem).wait()


_copy = pl.pallas_call(
    _copy_body,
    in_specs=[pl.BlockSpec(memory_space=pl.ANY)],
    out_specs=pl.BlockSpec(memory_space=pl.ANY),
    out_shape=jax.ShapeDtypeStruct((B, ROWS, LANES), jnp.float32),
    scratch_shapes=[pltpu.SemaphoreType.DMA],
)


@jax.jit
def kernel(inputs):
    x = inputs.reshape(B, A, SB, LANES, C)
    x = x.transpose(0, 1, 2, 4, 3).reshape(B, A, ROWS, LANES)
    out = _copy(x)
    out = out.reshape(B, SB, C, LANES).transpose(0, 1, 3, 2)
    return out.reshape(B, S, C)


# speedup vs baseline: 743.2459x; 10.9895x over previous
import jax
import jax.numpy as jnp
from jax.experimental import pallas as pl
from jax.experimental.pallas import tpu as pltpu

ACQ = 1
B, A, S, C = 64, 4, 4096, 2
LANES = 128
SB = S // LANES
ROWS = S * C // LANES

BB = 8


def _copy_body(in_ref, out_ref):
    out_ref[...] = in_ref[:, 0]


_copy = pl.pallas_call(
    _copy_body,
    grid=(B // BB,),
    in_specs=[
        pl.BlockSpec((BB, 1, ROWS, LANES), lambda i: (i, ACQ, 0, 0)),
    ],
    out_specs=pl.BlockSpec((BB, ROWS, LANES), lambda i: (i, 0, 0)),
    out_shape=jax.ShapeDtypeStruct((B, ROWS, LANES), jnp.float32),
)


@jax.jit
def kernel(inputs):
    x = inputs.reshape(B, A, SB, LANES, C)
    x = x.transpose(0, 1, 2, 4, 3).reshape(B, A, ROWS, LANES)
    out = _copy(x)
    out = out.reshape(B, SB, C, LANES).transpose(0, 1, 3, 2)
    return out.reshape(B, S, C)


# speedup vs baseline: 1646.6520x; 2.2155x over previous
import jax
import jax.numpy as jnp
from jax.experimental import pallas as pl
from jax.experimental.pallas import tpu as pltpu

ACQ = 1
B, A, S, C = 64, 4, 4096, 2
LANES = 128
SB = S // LANES
ROWS = S * C // LANES

CH = 8
CB = B // CH


def _copy_body(in_hbm, out_hbm, buf, insems, outsems):
    def in_copy(i):
        return pltpu.make_async_copy(
            in_hbm.at[pl.ds(i * CB, CB), ACQ],
            buf.at[pl.ds(i * CB, CB)],
            insems.at[i],
        )

    def out_copy(i):
        return pltpu.make_async_copy(
            buf.at[pl.ds(i * CB, CB)],
            out_hbm.at[pl.ds(i * CB, CB)],
            outsems.at[i],
        )

    for i in range(CH):
        in_copy(i).start()
    for i in range(CH):
        in_copy(i).wait()
        out_copy(i).start()
    for i in range(CH):
        out_copy(i).wait()


_copy = pl.pallas_call(
    _copy_body,
    in_specs=[pl.BlockSpec(memory_space=pl.ANY)],
    out_specs=pl.BlockSpec(memory_space=pl.ANY),
    out_shape=jax.ShapeDtypeStruct((B, ROWS, LANES), jnp.float32),
    scratch_shapes=[
        pltpu.VMEM((B, ROWS, LANES), jnp.float32),
        pltpu.SemaphoreType.DMA((CH,)),
        pltpu.SemaphoreType.DMA((CH,)),
    ],
)


@jax.jit
def kernel(inputs):
    x = inputs.reshape(B, A, SB, LANES, C)
    x = x.transpose(0, 1, 2, 4, 3).reshape(B, A, ROWS, LANES)
    out = _copy(x)
    out = out.reshape(B, SB, C, LANES).transpose(0, 1, 3, 2)
    return out.reshape(B, S, C)
